# Initial kernel scaffold; baseline (speedup 1.0000x reference)
import jax
import jax.numpy as jnp
from jax.experimental import pallas as pl


def kernel(tags, embed_table):
    raise NotImplementedError("write your pallas kernel here")


# speedup vs baseline: 2.5813x; 2.5813x over previous
import functools

import jax
import jax.numpy as jnp
from jax import lax
from jax.experimental import pallas as pl
from jax.experimental.pallas import tpu as pltpu
from jax.experimental.pallas import tpu_sc as plsc

TOPK = 50
ROW_BLOCK = 8
LANES = 128
NEG = -3.0e38
BIGI = 2**30


def _topk_body(ncols, tags_ref, idx_ref):
    main = (ncols // LANES) * LANES
    tail_w = ncols - main
    x = tags_ref[...]
    r = x.shape[0]
    y = x[:, :main].reshape(r, main // LANES, LANES)
    if tail_w:
        tail = x[:, main:]
        pad = jnp.full((r, LANES - tail_w), NEG, jnp.float32)
        tail_p = jnp.concatenate([tail, pad], axis=1)[:, None, :]
        y = jnp.concatenate([y, tail_p], axis=1)
    nchunk = y.shape[1]
    kchunks = min(TOPK, nchunk)

    cmax = jnp.max(y, axis=2)

    citer = lax.broadcasted_iota(jnp.int32, (r, nchunk), 1)
    cm = cmax
    id_cols = []
    for _ in range(kchunks):
        m = jnp.max(cm, axis=1, keepdims=True)
        pos = jnp.min(jnp.where(cm == m, citer, BIGI), axis=1, keepdims=True)
        cm = jnp.where(citer == pos, NEG, cm)
        id_cols.append(pos)
    cids = jnp.concatenate(id_cols, axis=1)

    chunk_iota = lax.broadcasted_iota(jnp.int32, (1, 1, nchunk), 2)
    oh = (cids[:, :, None] == chunk_iota).astype(jnp.float32)
    cand = lax.dot_general(oh, y, (((2,), (1,)), ((0,), (0,))),
                           precision=lax.Precision.HIGHEST,
                           preferred_element_type=jnp.float32)
    lane_iota = lax.broadcasted_iota(jnp.int32, (1, 1, LANES), 2)
    gidx = cids[:, :, None] * LANES + lane_iota
    cand = jnp.where(gidx >= ncols, NEG, cand)
    cv = cand.reshape(r, kchunks * LANES)
    gi = gidx.reshape(r, kchunks * LANES)

    out_cols = []
    for _ in range(TOPK):
        m = jnp.max(cv, axis=1, keepdims=True)
        pos = jnp.min(jnp.where(cv == m, gi, BIGI), axis=1, keepdims=True)
        cv = jnp.where(gi == pos, NEG, cv)
        out_cols.append(pos)
    idx_ref[...] = jnp.concatenate(out_cols, axis=1)


def _topk_idx(tags):
    nrows, ncols = tags.shape
    return pl.pallas_call(
        functools.partial(_topk_body, ncols),
        grid=(nrows // ROW_BLOCK,),
        in_specs=[pl.BlockSpec((ROW_BLOCK, ncols), lambda i: (i, 0))],
        out_specs=pl.BlockSpec((ROW_BLOCK, TOPK), lambda i: (i, 0)),
        out_shape=jax.ShapeDtypeStruct((nrows, TOPK), jnp.int32),
    )(tags)


def _sc_gather(table, idx_flat):
    info = plsc.get_sparse_core_info()
    nw = info.num_cores * info.num_subcores
    b = idx_flat.shape[0]
    d = table.shape[1]
    b_per_w = b // nw
    mesh = plsc.VectorSubcoreMesh(core_axis_name="c", subcore_axis_name="s")

    @functools.partial(
        pl.kernel, mesh=mesh,
        compiler_params=pltpu.CompilerParams(use_tc_tiling_on_sc=False),
        out_type=jax.ShapeDtypeStruct((b, d), jnp.float32),
        scratch_types=[
            pltpu.VMEM((b_per_w,), jnp.int32),
            pltpu.VMEM((b_per_w, d), jnp.float32),
            pltpu.SemaphoreType.DMA,
        ],
    )
    def k(table_hbm, idx_hbm, out_hbm, idx_v, rows_v, sem):
        wid = lax.axis_index("s") * info.num_cores + lax.axis_index("c")
        base = wid * b_per_w
        pltpu.sync_copy(idx_hbm.at[pl.ds(base, b_per_w)], idx_v)
        pltpu.async_copy(table_hbm.at[idx_v], rows_v, sem).wait()
        pltpu.sync_copy(rows_v, out_hbm.at[pl.ds(base, b_per_w)])

    return k(table, idx_flat)


def kernel(tags, embed_table):
    idx = _topk_idx(tags)
    rows = _sc_gather(embed_table, idx.reshape(-1))
    return rows.reshape(tags.shape[0], TOPK, embed_table.shape[1])


# speedup vs baseline: 4.6651x; 1.8073x over previous
import functools

import jax
import jax.numpy as jnp
from jax import lax
from jax.experimental import pallas as pl
from jax.experimental.pallas import tpu as pltpu
from jax.experimental.pallas import tpu_sc as plsc

TOPK = 50
ROW_BLOCK = 32
LANES = 128
NEG = -3.0e38
BIGI = 2**30


def _topk_body(ncols, yt_ref, idx_ref):
    y = yt_ref[...]
    r, _, nchunk = y.shape
    kchunks = min(TOPK, nchunk)

    cmax = jnp.max(y, axis=1)

    citer = lax.broadcasted_iota(jnp.int32, (r, nchunk), 1)
    cm = cmax
    id_cols = []
    for _ in range(kchunks):
        m = jnp.max(cm, axis=1, keepdims=True)
        pos = jnp.min(jnp.where(cm == m, citer, BIGI), axis=1, keepdims=True)
        cm = jnp.where(citer == pos, NEG, cm)
        id_cols.append(pos)
    cids = jnp.concatenate(id_cols, axis=1)

    chunk_iota = lax.broadcasted_iota(jnp.int32, (1, 1, nchunk), 2)
    oh = (cids[:, :, None] == chunk_iota).astype(jnp.float32)
    cand = lax.dot_general(oh, y, (((2,), (2,)), ((0,), (0,))),
                           precision=lax.Precision.HIGHEST,
                           preferred_element_type=jnp.float32)
    lane_iota = lax.broadcasted_iota(jnp.int32, (1, 1, LANES), 2)
    gidx = cids[:, :, None] * LANES + lane_iota
    cv = cand.reshape(r, kchunks * LANES)
    gi = gidx.reshape(r, kchunks * LANES)

    for t in range(TOPK):
        m = jnp.max(cv, axis=1, keepdims=True)
        pos = jnp.min(jnp.where(cv == m, gi, BIGI), axis=1, keepdims=True)
        cv = jnp.where(gi == pos, NEG, cv)
        idx_ref[:, t:t + 1] = pos


def _topk_idx(tags):
    nrows, ncols = tags.shape
    nchunk = (ncols + LANES - 1) // LANES
    pad = nchunk * LANES - ncols
    xp = jnp.pad(tags, ((0, 0), (0, pad)), constant_values=NEG)
    yt = xp.reshape(nrows, nchunk, LANES).transpose(0, 2, 1)
    return pl.pallas_call(
        functools.partial(_topk_body, ncols),
        grid=(nrows // ROW_BLOCK,),
        in_specs=[pl.BlockSpec((ROW_BLOCK, LANES, nchunk), lambda i: (i, 0, 0))],
        out_specs=pl.BlockSpec((ROW_BLOCK, TOPK), lambda i: (i, 0)),
        out_shape=jax.ShapeDtypeStruct((nrows, TOPK), jnp.int32),
    )(yt)


def _sc_gather(table, idx_flat):
    info = plsc.get_sparse_core_info()
    nw = info.num_cores * info.num_subcores
    b = idx_flat.shape[0]
    d = table.shape[1]
    b_per_w = b // nw
    mesh = plsc.VectorSubcoreMesh(core_axis_name="c", subcore_axis_name="s")

    @functools.partial(
        pl.kernel, mesh=mesh,
        compiler_params=pltpu.CompilerParams(use_tc_tiling_on_sc=False),
        out_type=jax.ShapeDtypeStruct((b, d), jnp.float32),
        scratch_types=[
            pltpu.VMEM((b_per_w,), jnp.int32),
            pltpu.VMEM((b_per_w, d), jnp.float32),
            pltpu.SemaphoreType.DMA,
        ],
    )
    def k(table_hbm, idx_hbm, out_hbm, idx_v, rows_v, sem):
        wid = lax.axis_index("s") * info.num_cores + lax.axis_index("c")
        base = wid * b_per_w
        pltpu.sync_copy(idx_hbm.at[pl.ds(base, b_per_w)], idx_v)
        pltpu.async_copy(table_hbm.at[idx_v], rows_v, sem).wait()
        pltpu.sync_copy(rows_v, out_hbm.at[pl.ds(base, b_per_w)])

    return k(table, idx_flat)


def kernel(tags, embed_table):
    idx = _topk_idx(tags)
    rows = _sc_gather(embed_table, idx.reshape(-1))
    return rows.reshape(tags.shape[0], TOPK, embed_table.shape[1])


# speedup vs baseline: 5.7778x; 1.2385x over previous
import functools

import jax
import jax.numpy as jnp
from jax import lax
from jax.experimental import pallas as pl
from jax.experimental.pallas import tpu as pltpu
from jax.experimental.pallas import tpu_sc as plsc

TOPK = 50
LANES = 128
RB_A = 32
RB_B = 128
NEG = -3.0e38
BIGI = 2**30


def _chunksel_body(yt_ref, cid_ref):
    y = yt_ref[...]
    r, _, nchunk = y.shape
    cm = jnp.max(y, axis=1)
    citer = lax.broadcasted_iota(jnp.int32, (r, nchunk), 1)
    for t in range(TOPK):
        m = jnp.max(cm, axis=1, keepdims=True)
        pos = jnp.min(jnp.where(cm == m, citer, BIGI), axis=1, keepdims=True)
        cm = jnp.where(citer == pos, NEG, cm)
        cid_ref[:, t:t + 1] = pos


def _extract_body(cv_ref, gi_ref, idx_ref):
    cv = cv_ref[...]
    gi = gi_ref[...]
    for t in range(TOPK):
        m = jnp.max(cv, axis=1, keepdims=True)
        pos = jnp.min(jnp.where(cv == m, gi, BIGI), axis=1, keepdims=True)
        cv = jnp.where(gi == pos, NEG, cv)
        idx_ref[:, t:t + 1] = pos


def _sc_gather(table, idx_flat):
    info = plsc.get_sparse_core_info()
    nw = info.num_cores * info.num_subcores
    b = idx_flat.shape[0]
    d = table.shape[1]
    b_per_w = b // nw
    nsplit = 1
    while (b_per_w // nsplit) * d * 4 > 420_000 or b_per_w % nsplit:
        nsplit += 1
    b_sub = b_per_w // nsplit
    mesh = plsc.VectorSubcoreMesh(core_axis_name="c", subcore_axis_name="s")

    @functools.partial(
        pl.kernel, mesh=mesh,
        compiler_params=pltpu.CompilerParams(use_tc_tiling_on_sc=False),
        out_type=jax.ShapeDtypeStruct((b, d), jnp.float32),
        scratch_types=[
            pltpu.VMEM((b_sub,), jnp.int32),
            pltpu.VMEM((b_sub, d), jnp.float32),
            pltpu.SemaphoreType.DMA,
        ],
    )
    def k(table_hbm, idx_hbm, out_hbm, idx_v, rows_v, sem):
        wid = lax.axis_index("s") * info.num_cores + lax.axis_index("c")
        for j in range(nsplit):
            base = wid * b_per_w + j * b_sub
            pltpu.sync_copy(idx_hbm.at[pl.ds(base, b_sub)], idx_v)
            pltpu.async_copy(table_hbm.at[idx_v], rows_v, sem).wait()
            pltpu.sync_copy(rows_v, out_hbm.at[pl.ds(base, b_sub)])

    return k(table, idx_flat)


def kernel(tags, embed_table):
    nrows, ncols = tags.shape
    nchunk = (ncols + LANES - 1) // LANES
    pad = nchunk * LANES - ncols
    xp = jnp.pad(tags, ((0, 0), (0, pad)), constant_values=NEG)
    yt = xp.reshape(nrows, nchunk, LANES).transpose(0, 2, 1)

    cids = pl.pallas_call(
        _chunksel_body,
        grid=(nrows // RB_A,),
        in_specs=[pl.BlockSpec((RB_A, LANES, nchunk), lambda i: (i, 0, 0))],
        out_specs=pl.BlockSpec((RB_A, TOPK), lambda i: (i, 0)),
        out_shape=jax.ShapeDtypeStruct((nrows, TOPK), jnp.int32),
    )(yt)

    chunk_rows = xp.reshape(nrows * nchunk, LANES)
    flat_cids = cids + jnp.arange(nrows, dtype=jnp.int32)[:, None] * nchunk
    cand = _sc_gather(chunk_rows, flat_cids.reshape(-1))
    cv = cand.reshape(nrows, TOPK * LANES)
    gi = (cids[:, :, None] * LANES
          + jnp.arange(LANES, dtype=jnp.int32)).reshape(nrows, TOPK * LANES)

    idx = pl.pallas_call(
        _extract_body,
        grid=(nrows // RB_B,),
        in_specs=[pl.BlockSpec((RB_B, TOPK * LANES), lambda i: (i, 0)),
                  pl.BlockSpec((RB_B, TOPK * LANES), lambda i: (i, 0))],
        out_specs=pl.BlockSpec((RB_B, TOPK), lambda i: (i, 0)),
        out_shape=jax.ShapeDtypeStruct((nrows, TOPK), jnp.int32),
    )(cv, gi)

    rows = _sc_gather(embed_table, idx.reshape(-1))
    return rows.reshape(nrows, TOPK, embed_table.shape[1])
